# Initial kernel scaffold; baseline (speedup 1.0000x reference)
import jax
import jax.numpy as jnp
from jax.experimental import pallas as pl


def kernel(x, edge_index, W1, b1, W2, b2, eps):
    raise NotImplementedError("write your pallas kernel here")


# speedup vs baseline: 2.9494x; 2.9494x over previous
import functools

import jax
import jax.numpy as jnp
from jax import lax
from jax.experimental import pallas as pl
from jax.experimental.pallas import tpu as pltpu
from jax.experimental.pallas import tpu_sc as plsc

IN_CH = 128
OUT_CH = 128
N_NODES = 10000
N_EDGES = 320000

CHUNK = 128
N_CHUNKS = N_EDGES // CHUNK
NC = 2
NS = 16
NW = NC * NS
CH_PER_W = -(-N_CHUNKS // NW)

NODE_BLK = 2000


def _tables_body(x_ref, w1_ref, eps_ref, a_ref, b_ref):
    xw = jnp.dot(x_ref[...], w1_ref[...], preferred_element_type=jnp.float32)
    b_ref[...] = xw
    a_ref[...] = (1.0 + eps_ref[0]) * xw


def _make_tables(x, W1, eps):
    grid = (N_NODES // NODE_BLK,)
    return pl.pallas_call(
        _tables_body,
        grid=grid,
        in_specs=[
            pl.BlockSpec((NODE_BLK, IN_CH), lambda i: (i, 0)),
            pl.BlockSpec((IN_CH, OUT_CH), lambda i: (0, 0)),
            pl.BlockSpec(memory_space=pltpu.SMEM),
        ],
        out_specs=[
            pl.BlockSpec((NODE_BLK, OUT_CH), lambda i: (i, 0)),
            pl.BlockSpec((NODE_BLK, OUT_CH), lambda i: (i, 0)),
        ],
        out_shape=[
            jax.ShapeDtypeStruct((N_NODES, OUT_CH), jnp.float32),
            jax.ShapeDtypeStruct((N_NODES, OUT_CH), jnp.float32),
        ],
    )(x, W1, eps)


_SC_MESH = plsc.VectorSubcoreMesh(core_axis_name="c", subcore_axis_name="s")


@functools.partial(
    pl.kernel,
    out_type=(
        jax.ShapeDtypeStruct((N_EDGES, OUT_CH), jnp.float32),
        jax.ShapeDtypeStruct((N_EDGES, OUT_CH), jnp.float32),
    ),
    mesh=_SC_MESH,
    scratch_types=[
        pltpu.VMEM((CHUNK,), jnp.int32),
        pltpu.VMEM((CHUNK,), jnp.int32),
        pltpu.VMEM((CHUNK, OUT_CH), jnp.float32),
        pltpu.VMEM((CHUNK, OUT_CH), jnp.float32),
        pltpu.SemaphoreType.DMA,
        pltpu.SemaphoreType.DMA,
    ],
)
def _sc_gather(a_hbm, b_hbm, row_hbm, col_hbm, g1_hbm, g2_hbm,
               idx_r, idx_c, rows_r, rows_c, sem_r, sem_c):
    wid = lax.axis_index("s") * NC + lax.axis_index("c")

    @pl.loop(0, CH_PER_W)
    def _(i):
        chunk = wid + i * NW

        @pl.when(chunk < N_CHUNKS)
        def _():
            base = chunk * CHUNK
            pltpu.sync_copy(row_hbm.at[pl.ds(base, CHUNK)], idx_r)
            pltpu.sync_copy(col_hbm.at[pl.ds(base, CHUNK)], idx_c)
            cp1 = pltpu.async_copy(a_hbm.at[idx_r], rows_r, sem_r)
            cp2 = pltpu.async_copy(b_hbm.at[idx_c], rows_c, sem_c)
            cp1.wait()
            cp2.wait()
            pltpu.sync_copy(rows_r, g1_hbm.at[pl.ds(base, CHUNK)])
            pltpu.sync_copy(rows_c, g2_hbm.at[pl.ds(base, CHUNK)])


EDGE_BLK = 3200


def _mlp_body(g1_ref, g2_ref, b1_ref, w2_ref, b2_ref, out_ref):
    h = jnp.maximum(g1_ref[...] + g2_ref[...] + b1_ref[...], 0.0)
    out_ref[...] = (
        jnp.dot(h, w2_ref[...], preferred_element_type=jnp.float32)
        + b2_ref[...]
    )


def _mlp(g1, g2, b1, W2, b2):
    grid = (N_EDGES // EDGE_BLK,)
    return pl.pallas_call(
        _mlp_body,
        grid=grid,
        in_specs=[
            pl.BlockSpec((EDGE_BLK, OUT_CH), lambda i: (i, 0)),
            pl.BlockSpec((EDGE_BLK, OUT_CH), lambda i: (i, 0)),
            pl.BlockSpec((1, OUT_CH), lambda i: (0, 0)),
            pl.BlockSpec((OUT_CH, OUT_CH), lambda i: (0, 0)),
            pl.BlockSpec((1, OUT_CH), lambda i: (0, 0)),
        ],
        out_specs=pl.BlockSpec((EDGE_BLK, OUT_CH), lambda i: (i, 0)),
        out_shape=jax.ShapeDtypeStruct((N_EDGES, OUT_CH), jnp.float32),
    )(g1, g2, b1.reshape(1, OUT_CH), W2, b2.reshape(1, OUT_CH))


def kernel(x, edge_index, W1, b1, W2, b2, eps):
    row = edge_index[0].astype(jnp.int32)
    col = edge_index[1].astype(jnp.int32)
    a_tab, b_tab = _make_tables(x, W1, eps)
    g1, g2 = _sc_gather(a_tab, b_tab, row, col)
    return _mlp(g1, g2, b1, W2, b2)


# speedup vs baseline: 3.8988x; 1.3219x over previous
import functools

import jax
import jax.numpy as jnp
from jax import lax
from jax.experimental import pallas as pl
from jax.experimental.pallas import tpu as pltpu
from jax.experimental.pallas import tpu_sc as plsc

IN_CH = 128
OUT_CH = 128
N_NODES = 10000
N_EDGES = 320000

CHUNK = 128
N_CHUNKS = N_EDGES // CHUNK
NC = 2
NS = 16
NW = NC * NS
CH_PER_W = 80
PAD_CHUNKS = CH_PER_W * NW

NODE_BLK = 2000


def _tables_body(x_ref, w1_ref, eps_ref, a_ref, b_ref):
    xw = jnp.dot(x_ref[...], w1_ref[...], preferred_element_type=jnp.float32)
    b_ref[...] = xw
    a_ref[...] = (1.0 + eps_ref[0]) * xw


def _make_tables(x, W1, eps):
    grid = (N_NODES // NODE_BLK,)
    return pl.pallas_call(
        _tables_body,
        grid=grid,
        in_specs=[
            pl.BlockSpec((NODE_BLK, IN_CH), lambda i: (i, 0)),
            pl.BlockSpec((IN_CH, OUT_CH), lambda i: (0, 0)),
            pl.BlockSpec(memory_space=pltpu.SMEM),
        ],
        out_specs=[
            pl.BlockSpec((NODE_BLK, OUT_CH), lambda i: (i, 0)),
            pl.BlockSpec((NODE_BLK, OUT_CH), lambda i: (i, 0)),
        ],
        out_shape=[
            jax.ShapeDtypeStruct((N_NODES, OUT_CH), jnp.float32),
            jax.ShapeDtypeStruct((N_NODES, OUT_CH), jnp.float32),
        ],
    )(x, W1, eps)


_SC_MESH = plsc.VectorSubcoreMesh(core_axis_name="c", subcore_axis_name="s")


@functools.partial(
    pl.kernel,
    out_type=(
        jax.ShapeDtypeStruct((N_EDGES, OUT_CH), jnp.float32),
        jax.ShapeDtypeStruct((N_EDGES, OUT_CH), jnp.float32),
    ),
    mesh=_SC_MESH,
    scratch_types=[
        pltpu.VMEM((CH_PER_W, CHUNK), jnp.int32),
        pltpu.VMEM((CH_PER_W, CHUNK), jnp.int32),
        pltpu.VMEM((CHUNK, OUT_CH), jnp.float32),
        pltpu.VMEM((CHUNK, OUT_CH), jnp.float32),
        pltpu.VMEM((CHUNK, OUT_CH), jnp.float32),
        pltpu.VMEM((CHUNK, OUT_CH), jnp.float32),
        pltpu.SemaphoreType.DMA,
        pltpu.SemaphoreType.DMA,
        pltpu.SemaphoreType.DMA,
        pltpu.SemaphoreType.DMA,
    ],
)
def _sc_gather(a_hbm, b_hbm, row_hbm, col_hbm, g1_hbm, g2_hbm,
               idx_r, idx_c, rows_r0, rows_r1, rows_c0, rows_c1,
               sem_r0, sem_r1, sem_c0, sem_c1):
    wid = lax.axis_index("s") * NC + lax.axis_index("c")
    base_chunk = wid * CH_PER_W
    rows_r = (rows_r0, rows_r1)
    rows_c = (rows_c0, rows_c1)
    sem_r = (sem_r0, sem_r1)
    sem_c = (sem_c0, sem_c1)

    pltpu.sync_copy(row_hbm.at[pl.ds(base_chunk, CH_PER_W)], idx_r)
    pltpu.sync_copy(col_hbm.at[pl.ds(base_chunk, CH_PER_W)], idx_c)

    @pl.loop(0, CH_PER_W + 2, step=2)
    def _(k0):
        for b in (0, 1):
            k = k0 + b
            chunk = base_chunk + k

            @pl.when((k < CH_PER_W) & (chunk < N_CHUNKS))
            def _():
                pltpu.async_copy(a_hbm.at[idx_r.at[k]], rows_r[b], sem_r[b])
                pltpu.async_copy(b_hbm.at[idx_c.at[k]], rows_c[b], sem_c[b])

            prev = chunk - 1

            @pl.when((k >= 1) & (k - 1 < CH_PER_W) & (prev < N_CHUNKS))
            def _():
                pb = 1 - b
                pltpu.make_async_copy(
                    a_hbm.at[idx_r.at[k - 1]], rows_r[pb], sem_r[pb]
                ).wait()
                pltpu.make_async_copy(
                    b_hbm.at[idx_c.at[k - 1]], rows_c[pb], sem_c[pb]
                ).wait()
                pltpu.sync_copy(
                    rows_r[pb], g1_hbm.at[pl.ds(prev * CHUNK, CHUNK)]
                )
                pltpu.sync_copy(
                    rows_c[pb], g2_hbm.at[pl.ds(prev * CHUNK, CHUNK)]
                )


EDGE_BLK = 3200


def _mlp_body(g1_ref, g2_ref, b1_ref, w2_ref, b2_ref, out_ref):
    h = jnp.maximum(
        g1_ref[...].astype(jnp.float32)
        + g2_ref[...].astype(jnp.float32)
        + b1_ref[...],
        0.0,
    )
    out_ref[...] = (
        jnp.dot(h, w2_ref[...], preferred_element_type=jnp.float32)
        + b2_ref[...]
    )


def _mlp(g1, g2, b1, W2, b2):
    grid = (N_EDGES // EDGE_BLK,)
    return pl.pallas_call(
        _mlp_body,
        grid=grid,
        in_specs=[
            pl.BlockSpec((EDGE_BLK, OUT_CH), lambda i: (i, 0)),
            pl.BlockSpec((EDGE_BLK, OUT_CH), lambda i: (i, 0)),
            pl.BlockSpec((1, OUT_CH), lambda i: (0, 0)),
            pl.BlockSpec((OUT_CH, OUT_CH), lambda i: (0, 0)),
            pl.BlockSpec((1, OUT_CH), lambda i: (0, 0)),
        ],
        out_specs=pl.BlockSpec((EDGE_BLK, OUT_CH), lambda i: (i, 0)),
        out_shape=jax.ShapeDtypeStruct((N_EDGES, OUT_CH), jnp.float32),
    )(g1, g2, b1.reshape(1, OUT_CH), W2, b2.reshape(1, OUT_CH))


def kernel(x, edge_index, W1, b1, W2, b2, eps):
    idx2d = edge_index.astype(jnp.int32).reshape(2, N_CHUNKS, CHUNK)
    pad = ((0, 0), (0, PAD_CHUNKS - N_CHUNKS), (0, 0))
    idx2d = jnp.pad(idx2d, pad)
    a_tab, b_tab = _make_tables(x, W1, eps)
    g1, g2 = _sc_gather(a_tab, b_tab, idx2d[0], idx2d[1])
    return _mlp(g1, g2, b1, W2, b2)
